# baseline (speedup 1.0000x reference)
import functools

import jax
import jax.numpy as jnp
from jax import lax
from jax.experimental import pallas as pl
from jax.experimental.pallas import tpu as pltpu
from jax.experimental.pallas import tpu_sc as plsc

N = 10000
E = 320000
NC = 2
NS = 16
CH = 125
NW = NC * NS
EPW = E // NW
NCH_A = EPW // CH
NCH_B = (E // NS) // CH
SEG = 4
CPS = NCH_B // SEG
NPA = 624
NREM = N - NS * NPA
DEGW = 16

BN = 1000
NB = N // BN

BN_SCALE = 1.0 / (1.0 + 1e-5) ** 0.5

_MESH = plsc.VectorSubcoreMesh(core_axis_name="c", subcore_axis_name="s")


def _per_tile_rows(s, do):
    do(pl.ds(s * NPA, NPA))

    @pl.when(s == NS - 1)
    def _():
        do(pl.ds(NS * NPA, NREM))


def _mp_pipelined(h_hbm, srcv, dstv, nch, rows, gsems, ssems, accsh):
    nbuf = len(rows)
    dummy = h_hbm.at[pl.ds(0, CH)]
    for b in range(nbuf):
        pltpu.async_copy(h_hbm.at[srcv.at[b]], rows[b], gsems[b])

    def round_(g, carry):
        for b in range(nbuf):
            j = nbuf * g + b
            bp = (b - 1) % nbuf

            def refill(j=j, bp=bp):
                pltpu.make_async_copy(dummy, rows[bp], ssems[bp]).wait()
                pltpu.async_copy(
                    h_hbm.at[srcv.at[jnp.minimum(j - 1 + nbuf, nch - 1)]],
                    rows[bp], gsems[bp])

            pltpu.make_async_copy(dummy, rows[b], gsems[b]).wait()
            pltpu.async_copy(rows[b], accsh.at[dstv.at[j]], ssems[b],
                             add=True)
            if b == 0:
                pl.when(g > 0)(refill)
            else:
                refill()
        return carry

    lax.fori_loop(0, nch // nbuf, round_, 0)
    for b in range(nbuf - 1):
        pltpu.make_async_copy(dummy, rows[b], gsems[b]).wait()
    pltpu.make_async_copy(dummy, rows[nbuf - 1], ssems[nbuf - 1]).wait()


def _deg_body(dst_hbm, ones_hbm, zeros_hbm, deg0_hbm, deg1_hbm,
              dstv, onesv, degsh):
    c = lax.axis_index("c")
    s = lax.axis_index("s")
    wid = 2 * s + c
    pltpu.sync_copy(dst_hbm.at[wid], dstv)
    pltpu.sync_copy(ones_hbm, onesv)
    _per_tile_rows(s, lambda sl: pltpu.sync_copy(zeros_hbm.at[sl],
                                                 degsh.at[sl]))
    plsc.subcore_barrier()

    def step(j, carry):
        pltpu.sync_copy(onesv, degsh.at[dstv.at[j]], add=True)
        return carry

    lax.fori_loop(0, NCH_A, step, 0)
    plsc.subcore_barrier()

    @pl.when(c == 0)
    def _():
        _per_tile_rows(s, lambda sl: pltpu.sync_copy(degsh.at[sl],
                                                     deg0_hbm.at[sl]))

    @pl.when(c == 1)
    def _():
        _per_tile_rows(s, lambda sl: pltpu.sync_copy(degsh.at[sl],
                                                     deg1_hbm.at[sl]))


_sc_deg = pl.kernel(
    _deg_body,
    out_type=(jax.ShapeDtypeStruct((N, DEGW), jnp.float32),
              jax.ShapeDtypeStruct((N, DEGW), jnp.float32)),
    mesh=_MESH,
    compiler_params=pltpu.CompilerParams(use_tc_tiling_on_sc=False),
    scratch_types=[
        pltpu.VMEM((NCH_A, CH), jnp.int32),
        pltpu.VMEM((CH, DEGW), jnp.float32),
        pltpu.VMEM_SHARED((N, DEGW), jnp.float32),
    ],
)


def _make_mp_edge(D):
    def body(h_hbm, src_hbm, dst_hbm, zeros_hbm, out0, out1,
             srcv, dstv, r0, r1, r2, r3, accsh,
             g0, g1, g2, g3, s0, s1, s2, s3):
        c = lax.axis_index("c")
        s = lax.axis_index("s")
        wid = 2 * s + c
        pltpu.sync_copy(src_hbm.at[wid], srcv)
        pltpu.sync_copy(dst_hbm.at[wid], dstv)
        _per_tile_rows(s, lambda sl: pltpu.sync_copy(zeros_hbm.at[sl],
                                                     accsh.at[sl]))
        plsc.subcore_barrier()
        _mp_pipelined(h_hbm, srcv, dstv, NCH_A, [r0, r1, r2, r3],
                      [g0, g1, g2, g3], [s0, s1, s2, s3], accsh)
        plsc.subcore_barrier()

        @pl.when(c == 0)
        def _():
            _per_tile_rows(s, lambda sl: pltpu.sync_copy(accsh.at[sl],
                                                         out0.at[sl]))

        @pl.when(c == 1)
        def _():
            _per_tile_rows(s, lambda sl: pltpu.sync_copy(accsh.at[sl],
                                                         out1.at[sl]))

    return pl.kernel(
        body,
        out_type=(jax.ShapeDtypeStruct((N, D), jnp.float32),
                  jax.ShapeDtypeStruct((N, D), jnp.float32)),
        mesh=_MESH,
        compiler_params=pltpu.CompilerParams(use_tc_tiling_on_sc=False),
        scratch_types=(
            [pltpu.VMEM((NCH_A, CH), jnp.int32),
             pltpu.VMEM((NCH_A, CH), jnp.int32)]
            + [pltpu.VMEM((CH, D), jnp.float32)] * 4
            + [pltpu.VMEM_SHARED((N, D), jnp.float32)]
            + [pltpu.SemaphoreType.DMA] * 8
        ),
    )


def _make_mp_feat(Dq):
    def body(hq0, hq1, hq2, hq3, src_hbm, dst_hbm, zeros_hbm,
             o0, o1, o2, o3,
             srcv, dstv, r0, r1, r2, r3, accsh,
             g0, g1, g2, g3, s0, s1, s2, s3):
        c = lax.axis_index("c")
        s = lax.axis_index("s")

        def phase(h_hbm, out_hbm):
            _per_tile_rows(s, lambda sl: pltpu.sync_copy(zeros_hbm.at[sl],
                                                         accsh.at[sl]))
            plsc.subcore_barrier()

            def seg_body(g, carry):
                pltpu.sync_copy(src_hbm.at[s * SEG + g], srcv)
                pltpu.sync_copy(dst_hbm.at[s * SEG + g], dstv)
                _mp_pipelined(h_hbm, srcv, dstv, CPS, [r0, r1, r2, r3],
                              [g0, g1, g2, g3], [s0, s1, s2, s3], accsh)
                return carry
            lax.fori_loop(0, SEG, seg_body, 0)
            plsc.subcore_barrier()
            _per_tile_rows(s, lambda sl: pltpu.sync_copy(accsh.at[sl],
                                                         out_hbm.at[sl]))
            plsc.subcore_barrier()

        @pl.when(c == 0)
        def _():
            phase(hq0, o0)
            phase(hq1, o1)

        @pl.when(c == 1)
        def _():
            phase(hq2, o2)
            phase(hq3, o3)

    return pl.kernel(
        body,
        out_type=tuple(jax.ShapeDtypeStruct((N, Dq), jnp.float32)
                       for _ in range(4)),
        mesh=_MESH,
        compiler_params=pltpu.CompilerParams(use_tc_tiling_on_sc=False),
        scratch_types=(
            [pltpu.VMEM((CPS, CH), jnp.int32),
             pltpu.VMEM((CPS, CH), jnp.int32)]
            + [pltpu.VMEM((CH, Dq), jnp.float32)] * 4
            + [pltpu.VMEM_SHARED((N, Dq), jnp.float32)]
            + [pltpu.SemaphoreType.DMA] * 8
        ),
    )


_mp96 = _make_mp_edge(96)
_mp64 = _make_mp_edge(64)
_mp_feat64 = _make_mp_feat(64)


def _prep1_body(x_ref, w_ref, d0_ref, d1_ref, o_ref, odinv_ref):
    dinv = lax.rsqrt(d0_ref[...][:, :1] + d1_ref[...][:, :1] + 1.0)
    h = jnp.dot(x_ref[...], w_ref[...], preferred_element_type=jnp.float32)
    o_ref[...] = h * dinv
    odinv_ref[...] = dinv


def _tc_prep1(x, W1, d0, d1):
    return pl.pallas_call(
        _prep1_body,
        grid=(NB,),
        in_specs=[
            pl.BlockSpec((BN, 128), lambda i: (i, 0)),
            pl.BlockSpec((128, 96), lambda i: (0, 0)),
            pl.BlockSpec((BN, DEGW), lambda i: (i, 0)),
            pl.BlockSpec((BN, DEGW), lambda i: (i, 0)),
        ],
        out_specs=[
            pl.BlockSpec((BN, 96), lambda i: (i, 0)),
            pl.BlockSpec((BN, 1), lambda i: (i, 0)),
        ],
        out_shape=[
            jax.ShapeDtypeStruct((N, 96), jnp.float32),
            jax.ShapeDtypeStruct((N, 1), jnp.float32),
        ],
    )(x, W1, d0, d1)


def _ep1_body(a0, a1, hp, dinv, b, g, be, w, o0, o1, o2, o3):
    pre = (a0[...] + a1[...] + hp[...]) * dinv[...] + b[...]
    h1 = jax.nn.relu(pre * (g[...] * BN_SCALE) + be[...])
    h2 = jnp.dot(h1, w[...], preferred_element_type=jnp.float32) * dinv[...]
    o0[...] = h2[:, 0:64]
    o1[...] = h2[:, 64:128]
    o2[...] = h2[:, 128:192]
    o3[...] = h2[:, 192:256]


def _tc_ep1(a0, a1, hp, dinv, b1, g1, be1, W2):
    return pl.pallas_call(
        _ep1_body,
        grid=(NB,),
        in_specs=[
            pl.BlockSpec((BN, 96), lambda i: (i, 0)),
            pl.BlockSpec((BN, 96), lambda i: (i, 0)),
            pl.BlockSpec((BN, 96), lambda i: (i, 0)),
            pl.BlockSpec((BN, 1), lambda i: (i, 0)),
            pl.BlockSpec((1, 96), lambda i: (0, 0)),
            pl.BlockSpec((1, 96), lambda i: (0, 0)),
            pl.BlockSpec((1, 96), lambda i: (0, 0)),
            pl.BlockSpec((96, 256), lambda i: (0, 0)),
        ],
        out_specs=[pl.BlockSpec((BN, 64), lambda i: (i, 0))] * 4,
        out_shape=[jax.ShapeDtypeStruct((N, 64), jnp.float32)] * 4,
    )(a0, a1, hp, dinv, b1.reshape(1, 96), g1.reshape(1, 96),
      be1.reshape(1, 96), W2)


def _ep2_body(a0, a1, a2, a3, hp0, hp1, hp2, hp3, dinv, b, g, be, w, o_ref):
    accf = jnp.concatenate([a0[...] + hp0[...], a1[...] + hp1[...],
                            a2[...] + hp2[...], a3[...] + hp3[...]], axis=1)
    pre = accf * dinv[...] + b[...]
    h2 = jax.nn.relu(pre * (g[...] * BN_SCALE) + be[...])
    o_ref[...] = jnp.dot(h2, w[...],
                         preferred_element_type=jnp.float32) * dinv[...]


def _tc_ep2(a0, a1, a2, a3, hp0, hp1, hp2, hp3, dinv, b2, g2, be2, W3):
    return pl.pallas_call(
        _ep2_body,
        grid=(NB,),
        in_specs=[
            pl.BlockSpec((BN, 64), lambda i: (i, 0)),
            pl.BlockSpec((BN, 64), lambda i: (i, 0)),
            pl.BlockSpec((BN, 64), lambda i: (i, 0)),
            pl.BlockSpec((BN, 64), lambda i: (i, 0)),
            pl.BlockSpec((BN, 64), lambda i: (i, 0)),
            pl.BlockSpec((BN, 64), lambda i: (i, 0)),
            pl.BlockSpec((BN, 64), lambda i: (i, 0)),
            pl.BlockSpec((BN, 64), lambda i: (i, 0)),
            pl.BlockSpec((BN, 1), lambda i: (i, 0)),
            pl.BlockSpec((1, 256), lambda i: (0, 0)),
            pl.BlockSpec((1, 256), lambda i: (0, 0)),
            pl.BlockSpec((1, 256), lambda i: (0, 0)),
            pl.BlockSpec((256, 64), lambda i: (0, 0)),
        ],
        out_specs=pl.BlockSpec((BN, 64), lambda i: (i, 0)),
        out_shape=jax.ShapeDtypeStruct((N, 64), jnp.float32),
    )(a0, a1, a2, a3, hp0, hp1, hp2, hp3, dinv, b2.reshape(1, 256),
      g2.reshape(1, 256), be2.reshape(1, 256), W3)


def _ep3_body(a0, a1, hp, dinv, b, g, be,
              wm1, bm1, wm2, bm2, wc1, bc1, wc2, bc2, om, oc):
    pre = (a0[...] + a1[...] + hp[...]) * dinv[...] + b[...]
    h3 = jax.nn.relu(pre * (g[...] * BN_SCALE) + be[...])
    hm = jax.nn.relu(jnp.dot(h3, wm1[...],
                             preferred_element_type=jnp.float32) + bm1[...])
    om[...] = jnp.dot(hm, wm2[...],
                      preferred_element_type=jnp.float32) + bm2[...]
    hc = jax.nn.relu(jnp.dot(h3, wc1[...],
                             preferred_element_type=jnp.float32) + bc1[...])
    oc[...] = jax.nn.sigmoid(
        jnp.dot(hc, wc2[...], preferred_element_type=jnp.float32) + bc2[...])


def _tc_ep3(a0, a1, hp, dinv, b3, g3, be3, Wm1, bm1, Wm2, bm2,
            Wc1, bc1, Wc2, bc2):
    def full(r, c):
        return pl.BlockSpec((r, c), lambda i: (0, 0))
    return pl.pallas_call(
        _ep3_body,
        grid=(NB,),
        in_specs=[
            pl.BlockSpec((BN, 64), lambda i: (i, 0)),
            pl.BlockSpec((BN, 64), lambda i: (i, 0)),
            pl.BlockSpec((BN, 64), lambda i: (i, 0)),
            pl.BlockSpec((BN, 1), lambda i: (i, 0)),
            full(1, 64), full(1, 64), full(1, 64),
            full(64, 32), full(1, 32), full(32, 1), full(1, 1),
            full(64, 32), full(1, 32), full(32, 1), full(1, 1),
        ],
        out_specs=[
            pl.BlockSpec((BN, 1), lambda i: (i, 0)),
            pl.BlockSpec((BN, 1), lambda i: (i, 0)),
        ],
        out_shape=[
            jax.ShapeDtypeStruct((N, 1), jnp.float32),
            jax.ShapeDtypeStruct((N, 1), jnp.float32),
        ],
    )(a0, a1, hp, dinv, b3.reshape(1, 64), g3.reshape(1, 64),
      be3.reshape(1, 64), Wm1, bm1.reshape(1, 32), Wm2, bm2.reshape(1, 1),
      Wc1, bc1.reshape(1, 32), Wc2, bc2.reshape(1, 1))


def kernel(x, edge_index, W1, b1, W2, b2, W3, b3, g1, be1, g2, be2, g3, be3,
           Wm1, bm1, Wm2, bm2, Wc1, bc1, Wc2, bc2):
    src = edge_index[0]
    dst = edge_index[1]
    src32 = src.reshape(NW, NCH_A, CH)
    dst32 = dst.reshape(NW, NCH_A, CH)
    src16 = src.reshape(NS * SEG, CPS, CH)
    dst16 = dst.reshape(NS * SEG, CPS, CH)

    ones_deg = jnp.ones((CH, DEGW), jnp.float32)
    d0, d1 = _sc_deg(dst32, ones_deg, jnp.zeros((N, DEGW), jnp.float32))

    h1p, dinv = _tc_prep1(x, W1, d0, d1)

    a0, a1 = _mp96(h1p, src32, dst32, jnp.zeros((N, 96), jnp.float32))
    q0, q1, q2, q3 = _tc_ep1(a0, a1, h1p, dinv, b1, g1, be1, W2)

    c0, c1, c2, c3 = _mp_feat64(q0, q1, q2, q3, src16, dst16,
                                jnp.zeros((N, 64), jnp.float32))
    h3p = _tc_ep2(c0, c1, c2, c3, q0, q1, q2, q3, dinv, b2, g2, be2, W3)

    e0, e1 = _mp64(h3p, src32, dst32, jnp.zeros((N, 64), jnp.float32))
    motor, cog = _tc_ep3(e0, e1, h3p, dinv, b3, g3, be3,
                         Wm1, bm1, Wm2, bm2, Wc1, bc1, Wc2, bc2)
    return motor.reshape(N), cog.reshape(N)


# speedup vs baseline: 1.0627x; 1.0627x over previous
import functools

import jax
import jax.numpy as jnp
from jax import lax
from jax.experimental import pallas as pl
from jax.experimental.pallas import tpu as pltpu
from jax.experimental.pallas import tpu_sc as plsc

N = 10000
E = 320000
NC = 2
NS = 16
CH = 125
NW = NC * NS
EPW = E // NW
NCH_A = EPW // CH
NPA = 624
NREM = N - NS * NPA
DEGW = 16

BN = 1000
NB = N // BN

BN_SCALE = 1.0 / (1.0 + 1e-5) ** 0.5

_MESH = plsc.VectorSubcoreMesh(core_axis_name="c", subcore_axis_name="s")


def _per_tile_rows(s, do):
    do(pl.ds(s * NPA, NPA))

    @pl.when(s == NS - 1)
    def _():
        do(pl.ds(NS * NPA, NREM))


def _fill(ref, nrow, ncol, value):
    val = jnp.full((16,), value, jnp.float32)

    def row(i, carry):
        for k in range(ncol // 16):
            ref[i, pl.ds(k * 16, 16)] = val
        return carry

    lax.fori_loop(0, nrow, row, 0)


def _zero_shared(s, zbuf, accsh, d):
    _fill(zbuf, CH, d, 0.0)

    def piece(k, carry):
        pltpu.sync_copy(zbuf.at[pl.ds(0, 104)],
                        accsh.at[pl.ds(s * NPA + k * 104, 104)])
        return carry

    lax.fori_loop(0, NPA // 104, piece, 0)

    @pl.when(s == NS - 1)
    def _():
        pltpu.sync_copy(zbuf.at[pl.ds(0, NREM)],
                        accsh.at[pl.ds(NS * NPA, NREM)])


def _mp_pipelined(h_hbm, srcv, dstv, nch, rows, gsems, ssems, accsh):
    nbuf = len(rows)
    dummy = h_hbm.at[pl.ds(0, CH)]
    for b in range(nbuf):
        pltpu.async_copy(h_hbm.at[srcv.at[b]], rows[b], gsems[b])

    def round_(g, carry):
        for b in range(nbuf):
            j = nbuf * g + b
            bp = (b - 1) % nbuf

            def refill(j=j, bp=bp):
                pltpu.make_async_copy(dummy, rows[bp], ssems[bp]).wait()
                pltpu.async_copy(
                    h_hbm.at[srcv.at[jnp.minimum(j - 1 + nbuf, nch - 1)]],
                    rows[bp], gsems[bp])

            pltpu.make_async_copy(dummy, rows[b], gsems[b]).wait()
            pltpu.async_copy(rows[b], accsh.at[dstv.at[j]], ssems[b],
                             add=True)
            if b == 0:
                pl.when(g > 0)(refill)
            else:
                refill()
        return carry

    lax.fori_loop(0, nch // nbuf, round_, 0)
    for b in range(nbuf - 1):
        pltpu.make_async_copy(dummy, rows[b], gsems[b]).wait()
    pltpu.make_async_copy(dummy, rows[nbuf - 1], ssems[nbuf - 1]).wait()


def _deg_body(ei_hbm, deg0_hbm, deg1_hbm, dstv, onesv, degsh):
    c = lax.axis_index("c")
    s = lax.axis_index("s")
    wid = 2 * s + c
    pltpu.sync_copy(ei_hbm.at[1, wid], dstv)
    _zero_shared(s, onesv, degsh, DEGW)
    _fill(onesv, CH, DEGW, 1.0)
    plsc.subcore_barrier()

    def step(j, carry):
        pltpu.sync_copy(onesv, degsh.at[dstv.at[j]], add=True)
        return carry

    lax.fori_loop(0, NCH_A, step, 0)
    plsc.subcore_barrier()

    @pl.when(c == 0)
    def _():
        _per_tile_rows(s, lambda sl: pltpu.sync_copy(degsh.at[sl],
                                                     deg0_hbm.at[sl]))

    @pl.when(c == 1)
    def _():
        _per_tile_rows(s, lambda sl: pltpu.sync_copy(degsh.at[sl],
                                                     deg1_hbm.at[sl]))


_sc_deg = pl.kernel(
    _deg_body,
    out_type=(jax.ShapeDtypeStruct((N, DEGW), jnp.float32),
              jax.ShapeDtypeStruct((N, DEGW), jnp.float32)),
    mesh=_MESH,
    compiler_params=pltpu.CompilerParams(use_tc_tiling_on_sc=False),
    scratch_types=[
        pltpu.VMEM((NCH_A, CH), jnp.int32),
        pltpu.VMEM((CH, DEGW), jnp.float32),
        pltpu.VMEM_SHARED((N, DEGW), jnp.float32),
    ],
)


def _make_mp_edge(D):
    def body(h_hbm, ei_hbm, out0, out1,
             srcv, dstv, r0, r1, r2, r3, accsh,
             g0, g1, g2, g3, s0, s1, s2, s3):
        c = lax.axis_index("c")
        s = lax.axis_index("s")
        wid = 2 * s + c
        pltpu.sync_copy(ei_hbm.at[0, wid], srcv)
        pltpu.sync_copy(ei_hbm.at[1, wid], dstv)
        _zero_shared(s, r0, accsh, D)
        plsc.subcore_barrier()
        _mp_pipelined(h_hbm, srcv, dstv, NCH_A, [r0, r1, r2, r3],
                      [g0, g1, g2, g3], [s0, s1, s2, s3], accsh)
        plsc.subcore_barrier()

        @pl.when(c == 0)
        def _():
            _per_tile_rows(s, lambda sl: pltpu.sync_copy(accsh.at[sl],
                                                         out0.at[sl]))

        @pl.when(c == 1)
        def _():
            _per_tile_rows(s, lambda sl: pltpu.sync_copy(accsh.at[sl],
                                                         out1.at[sl]))

    return pl.kernel(
        body,
        out_type=(jax.ShapeDtypeStruct((N, D), jnp.float32),
                  jax.ShapeDtypeStruct((N, D), jnp.float32)),
        mesh=_MESH,
        compiler_params=pltpu.CompilerParams(use_tc_tiling_on_sc=False),
        scratch_types=(
            [pltpu.VMEM((NCH_A, CH), jnp.int32),
             pltpu.VMEM((NCH_A, CH), jnp.int32)]
            + [pltpu.VMEM((CH, D), jnp.float32)] * 4
            + [pltpu.VMEM_SHARED((N, D), jnp.float32)]
            + [pltpu.SemaphoreType.DMA] * 8
        ),
    )


def _make_mp_feat(Dq):
    def body(hq0, hq1, hq2, hq3, ei_hbm,
             o0, o1, o2, o3,
             srcv, dstv, r0, r1, r2, r3, accsh,
             g0, g1, g2, g3, s0, s1, s2, s3):
        c = lax.axis_index("c")
        s = lax.axis_index("s")

        def phase(h_hbm, out_hbm):
            _zero_shared(s, r0, accsh, Dq)
            plsc.subcore_barrier()

            for q in range(2):
                pltpu.sync_copy(ei_hbm.at[0, 2 * s + q], srcv)
                pltpu.sync_copy(ei_hbm.at[1, 2 * s + q], dstv)
                _mp_pipelined(h_hbm, srcv, dstv, NCH_A, [r0, r1, r2, r3],
                              [g0, g1, g2, g3], [s0, s1, s2, s3], accsh)
            plsc.subcore_barrier()
            _per_tile_rows(s, lambda sl: pltpu.sync_copy(accsh.at[sl],
                                                         out_hbm.at[sl]))
            plsc.subcore_barrier()

        @pl.when(c == 0)
        def _():
            phase(hq0, o0)
            phase(hq1, o1)

        @pl.when(c == 1)
        def _():
            phase(hq2, o2)
            phase(hq3, o3)

    return pl.kernel(
        body,
        out_type=tuple(jax.ShapeDtypeStruct((N, Dq), jnp.float32)
                       for _ in range(4)),
        mesh=_MESH,
        compiler_params=pltpu.CompilerParams(use_tc_tiling_on_sc=False),
        scratch_types=(
            [pltpu.VMEM((NCH_A, CH), jnp.int32),
             pltpu.VMEM((NCH_A, CH), jnp.int32)]
            + [pltpu.VMEM((CH, Dq), jnp.float32)] * 4
            + [pltpu.VMEM_SHARED((N, Dq), jnp.float32)]
            + [pltpu.SemaphoreType.DMA] * 8
        ),
    )


_mp96 = _make_mp_edge(96)
_mp64 = _make_mp_edge(64)
_mp_feat64 = _make_mp_feat(64)


def _prep1_body(x_ref, w_ref, d0_ref, d1_ref, o_ref, odinv_ref):
    dinv = lax.rsqrt(d0_ref[...][:, :1] + d1_ref[...][:, :1] + 1.0)
    h = jnp.dot(x_ref[...], w_ref[...], preferred_element_type=jnp.float32)
    o_ref[...] = h * dinv
    odinv_ref[...] = dinv


def _tc_prep1(x, W1, d0, d1):
    return pl.pallas_call(
        _prep1_body,
        grid=(NB,),
        in_specs=[
            pl.BlockSpec((BN, 128), lambda i: (i, 0)),
            pl.BlockSpec((128, 96), lambda i: (0, 0)),
            pl.BlockSpec((BN, DEGW), lambda i: (i, 0)),
            pl.BlockSpec((BN, DEGW), lambda i: (i, 0)),
        ],
        out_specs=[
            pl.BlockSpec((BN, 96), lambda i: (i, 0)),
            pl.BlockSpec((BN, 1), lambda i: (i, 0)),
        ],
        out_shape=[
            jax.ShapeDtypeStruct((N, 96), jnp.float32),
            jax.ShapeDtypeStruct((N, 1), jnp.float32),
        ],
    )(x, W1, d0, d1)


def _ep1_body(a0, a1, hp, dinv, b, g, be, w, o0, o1, o2, o3):
    pre = (a0[...] + a1[...] + hp[...]) * dinv[...] + b[...]
    h1 = jax.nn.relu(pre * (g[...] * BN_SCALE) + be[...])
    h2 = jnp.dot(h1, w[...], preferred_element_type=jnp.float32) * dinv[...]
    o0[...] = h2[:, 0:64]
    o1[...] = h2[:, 64:128]
    o2[...] = h2[:, 128:192]
    o3[...] = h2[:, 192:256]


def _tc_ep1(a0, a1, hp, dinv, b1, g1, be1, W2):
    return pl.pallas_call(
        _ep1_body,
        grid=(NB,),
        in_specs=[
            pl.BlockSpec((BN, 96), lambda i: (i, 0)),
            pl.BlockSpec((BN, 96), lambda i: (i, 0)),
            pl.BlockSpec((BN, 96), lambda i: (i, 0)),
            pl.BlockSpec((BN, 1), lambda i: (i, 0)),
            pl.BlockSpec((1, 96), lambda i: (0, 0)),
            pl.BlockSpec((1, 96), lambda i: (0, 0)),
            pl.BlockSpec((1, 96), lambda i: (0, 0)),
            pl.BlockSpec((96, 256), lambda i: (0, 0)),
        ],
        out_specs=[pl.BlockSpec((BN, 64), lambda i: (i, 0))] * 4,
        out_shape=[jax.ShapeDtypeStruct((N, 64), jnp.float32)] * 4,
    )(a0, a1, hp, dinv, b1.reshape(1, 96), g1.reshape(1, 96),
      be1.reshape(1, 96), W2)


def _ep2_body(a0, a1, a2, a3, hp0, hp1, hp2, hp3, dinv, b, g, be, w, o_ref):
    accf = jnp.concatenate([a0[...] + hp0[...], a1[...] + hp1[...],
                            a2[...] + hp2[...], a3[...] + hp3[...]], axis=1)
    pre = accf * dinv[...] + b[...]
    h2 = jax.nn.relu(pre * (g[...] * BN_SCALE) + be[...])
    o_ref[...] = jnp.dot(h2, w[...],
                         preferred_element_type=jnp.float32) * dinv[...]


def _tc_ep2(a0, a1, a2, a3, hp0, hp1, hp2, hp3, dinv, b2, g2, be2, W3):
    return pl.pallas_call(
        _ep2_body,
        grid=(NB,),
        in_specs=[
            pl.BlockSpec((BN, 64), lambda i: (i, 0)),
            pl.BlockSpec((BN, 64), lambda i: (i, 0)),
            pl.BlockSpec((BN, 64), lambda i: (i, 0)),
            pl.BlockSpec((BN, 64), lambda i: (i, 0)),
            pl.BlockSpec((BN, 64), lambda i: (i, 0)),
            pl.BlockSpec((BN, 64), lambda i: (i, 0)),
            pl.BlockSpec((BN, 64), lambda i: (i, 0)),
            pl.BlockSpec((BN, 64), lambda i: (i, 0)),
            pl.BlockSpec((BN, 1), lambda i: (i, 0)),
            pl.BlockSpec((1, 256), lambda i: (0, 0)),
            pl.BlockSpec((1, 256), lambda i: (0, 0)),
            pl.BlockSpec((1, 256), lambda i: (0, 0)),
            pl.BlockSpec((256, 64), lambda i: (0, 0)),
        ],
        out_specs=pl.BlockSpec((BN, 64), lambda i: (i, 0)),
        out_shape=jax.ShapeDtypeStruct((N, 64), jnp.float32),
    )(a0, a1, a2, a3, hp0, hp1, hp2, hp3, dinv, b2.reshape(1, 256),
      g2.reshape(1, 256), be2.reshape(1, 256), W3)


def _ep3_body(a0, a1, hp, dinv, b, g, be,
              wm1, bm1, wm2, bm2, wc1, bc1, wc2, bc2, om, oc):
    pre = (a0[...] + a1[...] + hp[...]) * dinv[...] + b[...]
    h3 = jax.nn.relu(pre * (g[...] * BN_SCALE) + be[...])
    hm = jax.nn.relu(jnp.dot(h3, wm1[...],
                             preferred_element_type=jnp.float32) + bm1[...])
    om[...] = jnp.dot(hm, wm2[...],
                      preferred_element_type=jnp.float32) + bm2[...]
    hc = jax.nn.relu(jnp.dot(h3, wc1[...],
                             preferred_element_type=jnp.float32) + bc1[...])
    oc[...] = jax.nn.sigmoid(
        jnp.dot(hc, wc2[...], preferred_element_type=jnp.float32) + bc2[...])


def _tc_ep3(a0, a1, hp, dinv, b3, g3, be3, Wm1, bm1, Wm2, bm2,
            Wc1, bc1, Wc2, bc2):
    def full(r, c):
        return pl.BlockSpec((r, c), lambda i: (0, 0))
    return pl.pallas_call(
        _ep3_body,
        grid=(NB,),
        in_specs=[
            pl.BlockSpec((BN, 64), lambda i: (i, 0)),
            pl.BlockSpec((BN, 64), lambda i: (i, 0)),
            pl.BlockSpec((BN, 64), lambda i: (i, 0)),
            pl.BlockSpec((BN, 1), lambda i: (i, 0)),
            full(1, 64), full(1, 64), full(1, 64),
            full(64, 32), full(1, 32), full(32, 1), full(1, 1),
            full(64, 32), full(1, 32), full(32, 1), full(1, 1),
        ],
        out_specs=[
            pl.BlockSpec((BN, 1), lambda i: (i, 0)),
            pl.BlockSpec((BN, 1), lambda i: (i, 0)),
        ],
        out_shape=[
            jax.ShapeDtypeStruct((N, 1), jnp.float32),
            jax.ShapeDtypeStruct((N, 1), jnp.float32),
        ],
    )(a0, a1, hp, dinv, b3.reshape(1, 64), g3.reshape(1, 64),
      be3.reshape(1, 64), Wm1, bm1.reshape(1, 32), Wm2, bm2.reshape(1, 1),
      Wc1, bc1.reshape(1, 32), Wc2, bc2.reshape(1, 1))


def kernel(x, edge_index, W1, b1, W2, b2, W3, b3, g1, be1, g2, be2, g3, be3,
           Wm1, bm1, Wm2, bm2, Wc1, bc1, Wc2, bc2):
    ei4 = edge_index.reshape(2, NW, NCH_A, CH)

    d0, d1 = _sc_deg(ei4)

    h1p, dinv = _tc_prep1(x, W1, d0, d1)

    a0, a1 = _mp96(h1p, ei4)
    q0, q1, q2, q3 = _tc_ep1(a0, a1, h1p, dinv, b1, g1, be1, W2)

    c0, c1, c2, c3 = _mp_feat64(q0, q1, q2, q3, ei4)
    h3p = _tc_ep2(c0, c1, c2, c3, q0, q1, q2, q3, dinv, b2, g2, be2, W3)

    e0, e1 = _mp64(h3p, ei4)
    motor, cog = _tc_ep3(e0, e1, h3p, dinv, b3, g3, be3,
                         Wm1, bm1, Wm2, bm2, Wc1, bc1, Wc2, bc2)
    return motor.reshape(N), cog.reshape(N)


# speedup vs baseline: 1.0753x; 1.0118x over previous
import functools

import jax
import jax.numpy as jnp
from jax import lax
from jax.experimental import pallas as pl
from jax.experimental.pallas import tpu as pltpu
from jax.experimental.pallas import tpu_sc as plsc

N = 10000
E = 320000
NC = 2
NS = 16
CH = 125
NW = NC * NS
EPW = E // NW
NCH_A = EPW // CH
NPA = 624
NREM = N - NS * NPA
DEGW = 16

BN = 1000
NB = N // BN

BN_SCALE = 1.0 / (1.0 + 1e-5) ** 0.5

_MESH = plsc.VectorSubcoreMesh(core_axis_name="c", subcore_axis_name="s")


def _per_tile_rows(s, do):
    do(pl.ds(s * NPA, NPA))

    @pl.when(s == NS - 1)
    def _():
        do(pl.ds(NS * NPA, NREM))


def _fill(ref, nrow, ncol, value):
    val = jnp.full((16,), value, jnp.float32)

    def row(i, carry):
        for k in range(ncol // 16):
            ref[i, pl.ds(k * 16, 16)] = val
        return carry

    lax.fori_loop(0, nrow, row, 0)


def _zero_shared(s, zbuf, accsh, d):
    _fill(zbuf, CH, d, 0.0)

    def piece(k, carry):
        pltpu.sync_copy(zbuf.at[pl.ds(0, 104)],
                        accsh.at[pl.ds(s * NPA + k * 104, 104)])
        return carry

    lax.fori_loop(0, NPA // 104, piece, 0)

    @pl.when(s == NS - 1)
    def _():
        pltpu.sync_copy(zbuf.at[pl.ds(0, NREM)],
                        accsh.at[pl.ds(NS * NPA, NREM)])


def _mp_pipelined(h_hbm, srcv, dstv, nch, rows, gsems, ssems, accsh):
    nbuf = len(rows)
    dummy = h_hbm.at[pl.ds(0, CH)]
    for b in range(nbuf):
        pltpu.async_copy(h_hbm.at[srcv.at[b]], rows[b], gsems[b])

    def round_(g, carry):
        for b in range(nbuf):
            j = nbuf * g + b
            bp = (b - 1) % nbuf

            def refill(j=j, bp=bp):
                pltpu.make_async_copy(dummy, rows[bp], ssems[bp]).wait()
                pltpu.async_copy(
                    h_hbm.at[srcv.at[jnp.minimum(j - 1 + nbuf, nch - 1)]],
                    rows[bp], gsems[bp])

            pltpu.make_async_copy(dummy, rows[b], gsems[b]).wait()
            pltpu.async_copy(rows[b], accsh.at[dstv.at[j]], ssems[b],
                             add=True)
            if b == 0:
                pl.when(g > 0)(refill)
            else:
                refill()
        return carry

    lax.fori_loop(0, nch // nbuf, round_, 0)
    for b in range(nbuf - 1):
        pltpu.make_async_copy(dummy, rows[b], gsems[b]).wait()
    pltpu.make_async_copy(dummy, rows[nbuf - 1], ssems[nbuf - 1]).wait()


def _deg_body(ei_hbm, deg0_hbm, deg1_hbm, dstv, onesv, degsh):
    c = lax.axis_index("c")
    s = lax.axis_index("s")
    wid = 2 * s + c
    pltpu.sync_copy(ei_hbm.at[1, wid], dstv)
    _zero_shared(s, onesv, degsh, DEGW)
    _fill(onesv, CH, DEGW, 1.0)
    plsc.subcore_barrier()

    def step(j, carry):
        pltpu.sync_copy(onesv, degsh.at[dstv.at[j]], add=True)
        return carry

    lax.fori_loop(0, NCH_A, step, 0)
    plsc.subcore_barrier()

    @pl.when(c == 0)
    def _():
        _per_tile_rows(s, lambda sl: pltpu.sync_copy(degsh.at[sl],
                                                     deg0_hbm.at[sl]))

    @pl.when(c == 1)
    def _():
        _per_tile_rows(s, lambda sl: pltpu.sync_copy(degsh.at[sl],
                                                     deg1_hbm.at[sl]))


_sc_deg = pl.kernel(
    _deg_body,
    out_type=(jax.ShapeDtypeStruct((N, DEGW), jnp.float32),
              jax.ShapeDtypeStruct((N, DEGW), jnp.float32)),
    mesh=_MESH,
    compiler_params=pltpu.CompilerParams(use_tc_tiling_on_sc=False),
    scratch_types=[
        pltpu.VMEM((NCH_A, CH), jnp.int32),
        pltpu.VMEM((CH, DEGW), jnp.float32),
        pltpu.VMEM_SHARED((N, DEGW), jnp.float32),
    ],
)


def _make_mp_edge(D):
    def body(h_hbm, ei_hbm, out0, out1,
             srcv, dstv, r0, r1, r2, r3, accsh,
             g0, g1, g2, g3, s0, s1, s2, s3):
        c = lax.axis_index("c")
        s = lax.axis_index("s")
        wid = 2 * s + c
        pltpu.sync_copy(ei_hbm.at[0, wid], srcv)
        pltpu.sync_copy(ei_hbm.at[1, wid], dstv)
        _zero_shared(s, r0, accsh, D)
        plsc.subcore_barrier()
        _mp_pipelined(h_hbm, srcv, dstv, NCH_A, [r0, r1, r2, r3],
                      [g0, g1, g2, g3], [s0, s1, s2, s3], accsh)
        plsc.subcore_barrier()

        @pl.when(c == 0)
        def _():
            _per_tile_rows(s, lambda sl: pltpu.sync_copy(accsh.at[sl],
                                                         out0.at[sl]))

        @pl.when(c == 1)
        def _():
            _per_tile_rows(s, lambda sl: pltpu.sync_copy(accsh.at[sl],
                                                         out1.at[sl]))

    return pl.kernel(
        body,
        out_type=(jax.ShapeDtypeStruct((N, D), jnp.float32),
                  jax.ShapeDtypeStruct((N, D), jnp.float32)),
        mesh=_MESH,
        compiler_params=pltpu.CompilerParams(use_tc_tiling_on_sc=False),
        scratch_types=(
            [pltpu.VMEM((NCH_A, CH), jnp.int32),
             pltpu.VMEM((NCH_A, CH), jnp.int32)]
            + [pltpu.VMEM((CH, D), jnp.float32)] * 4
            + [pltpu.VMEM_SHARED((N, D), jnp.float32)]
            + [pltpu.SemaphoreType.DMA] * 8
        ),
    )


def _make_mp_feat(Dq):
    def body(hq0, hq1, hq2, hq3, ei_hbm,
             o0, o1, o2, o3,
             srcv, dstv, r0, r1, r2, r3, accsh,
             g0, g1, g2, g3, s0, s1, s2, s3):
        c = lax.axis_index("c")
        s = lax.axis_index("s")

        def phase(h_hbm, out_hbm):
            _zero_shared(s, r0, accsh, Dq)
            plsc.subcore_barrier()

            for q in range(2):
                pltpu.sync_copy(ei_hbm.at[0, 2 * s + q], srcv)
                pltpu.sync_copy(ei_hbm.at[1, 2 * s + q], dstv)
                _mp_pipelined(h_hbm, srcv, dstv, NCH_A, [r0, r1, r2, r3],
                              [g0, g1, g2, g3], [s0, s1, s2, s3], accsh)
            plsc.subcore_barrier()
            _per_tile_rows(s, lambda sl: pltpu.sync_copy(accsh.at[sl],
                                                         out_hbm.at[sl]))
            plsc.subcore_barrier()

        @pl.when(c == 0)
        def _():
            phase(hq0, o0)
            phase(hq1, o1)

        @pl.when(c == 1)
        def _():
            phase(hq2, o2)
            phase(hq3, o3)

    return pl.kernel(
        body,
        out_type=tuple(jax.ShapeDtypeStruct((N, Dq), jnp.float32)
                       for _ in range(4)),
        mesh=_MESH,
        compiler_params=pltpu.CompilerParams(use_tc_tiling_on_sc=False),
        scratch_types=(
            [pltpu.VMEM((NCH_A, CH), jnp.int32),
             pltpu.VMEM((NCH_A, CH), jnp.int32)]
            + [pltpu.VMEM((CH, Dq), jnp.float32)] * 4
            + [pltpu.VMEM_SHARED((N, Dq), jnp.float32)]
            + [pltpu.SemaphoreType.DMA] * 8
        ),
    )


_mp96 = _make_mp_edge(96)
_mp64 = _make_mp_edge(64)
_mp_feat64 = _make_mp_feat(64)


def _prep1_body(x_ref, w_ref, d0_ref, d1_ref, o_ref, odinv_ref):
    dinv = lax.rsqrt(d0_ref[...][:, :1] + d1_ref[...][:, :1] + 1.0)
    h = jnp.dot(x_ref[...], w_ref[...], preferred_element_type=jnp.float32)
    o_ref[...] = h * dinv
    odinv_ref[...] = dinv


def _tc_prep1(x, W1, d0, d1):
    return pl.pallas_call(
        _prep1_body,
        grid=(NB,),
        in_specs=[
            pl.BlockSpec((BN, 128), lambda i: (i, 0)),
            pl.BlockSpec((128, 96), lambda i: (0, 0)),
            pl.BlockSpec((BN, DEGW), lambda i: (i, 0)),
            pl.BlockSpec((BN, DEGW), lambda i: (i, 0)),
        ],
        out_specs=[
            pl.BlockSpec((BN, 96), lambda i: (i, 0)),
            pl.BlockSpec((BN, 1), lambda i: (i, 0)),
        ],
        out_shape=[
            jax.ShapeDtypeStruct((N, 96), jnp.float32),
            jax.ShapeDtypeStruct((N, 1), jnp.float32),
        ],
    )(x, W1, d0, d1)


def _ep1_body(a0, a1, hp, dinv, b, g, be, w, o0, o1, o2, o3):
    pre = (a0[...] + a1[...] + hp[...]) * dinv[...] + b[...]
    h1 = jax.nn.relu(pre * (g[...] * BN_SCALE) + be[...])
    h2 = jnp.dot(h1, w[...], preferred_element_type=jnp.float32) * dinv[...]
    o0[...] = h2[:, 0:64]
    o1[...] = h2[:, 64:128]
    o2[...] = h2[:, 128:192]
    o3[...] = h2[:, 192:256]


def _tc_ep1(a0, a1, hp, dinv, b1, g1, be1, W2):
    return pl.pallas_call(
        _ep1_body,
        grid=(NB,),
        in_specs=[
            pl.BlockSpec((BN, 96), lambda i: (i, 0)),
            pl.BlockSpec((BN, 96), lambda i: (i, 0)),
            pl.BlockSpec((BN, 96), lambda i: (i, 0)),
            pl.BlockSpec((BN, 1), lambda i: (i, 0)),
            pl.BlockSpec((1, 96), lambda i: (0, 0)),
            pl.BlockSpec((1, 96), lambda i: (0, 0)),
            pl.BlockSpec((1, 96), lambda i: (0, 0)),
            pl.BlockSpec((96, 256), lambda i: (0, 0)),
        ],
        out_specs=[pl.BlockSpec((BN, 64), lambda i: (i, 0))] * 4,
        out_shape=[jax.ShapeDtypeStruct((N, 64), jnp.float32)] * 4,
    )(a0, a1, hp, dinv, b1.reshape(1, 96), g1.reshape(1, 96),
      be1.reshape(1, 96), W2)


def _ep2_body(a0, a1, a2, a3, hp0, hp1, hp2, hp3, dinv, b, g, be, w, o_ref):
    accf = jnp.concatenate([a0[...] + hp0[...], a1[...] + hp1[...],
                            a2[...] + hp2[...], a3[...] + hp3[...]], axis=1)
    pre = accf * dinv[...] + b[...]
    h2 = jax.nn.relu(pre * (g[...] * BN_SCALE) + be[...])
    o_ref[...] = jnp.dot(h2, w[...],
                         preferred_element_type=jnp.float32) * dinv[...]


def _tc_ep2(a0, a1, a2, a3, hp0, hp1, hp2, hp3, dinv, b2, g2, be2, W3):
    return pl.pallas_call(
        _ep2_body,
        grid=(NB,),
        in_specs=[
            pl.BlockSpec((BN, 64), lambda i: (i, 0)),
            pl.BlockSpec((BN, 64), lambda i: (i, 0)),
            pl.BlockSpec((BN, 64), lambda i: (i, 0)),
            pl.BlockSpec((BN, 64), lambda i: (i, 0)),
            pl.BlockSpec((BN, 64), lambda i: (i, 0)),
            pl.BlockSpec((BN, 64), lambda i: (i, 0)),
            pl.BlockSpec((BN, 64), lambda i: (i, 0)),
            pl.BlockSpec((BN, 64), lambda i: (i, 0)),
            pl.BlockSpec((BN, 1), lambda i: (i, 0)),
            pl.BlockSpec((1, 256), lambda i: (0, 0)),
            pl.BlockSpec((1, 256), lambda i: (0, 0)),
            pl.BlockSpec((1, 256), lambda i: (0, 0)),
            pl.BlockSpec((256, 64), lambda i: (0, 0)),
        ],
        out_specs=pl.BlockSpec((BN, 64), lambda i: (i, 0)),
        out_shape=jax.ShapeDtypeStruct((N, 64), jnp.float32),
    )(a0, a1, a2, a3, hp0, hp1, hp2, hp3, dinv, b2.reshape(1, 256),
      g2.reshape(1, 256), be2.reshape(1, 256), W3)


def _ep3_body(a0, a1, hp, dinv, b, g, be,
              wm1, bm1, wm2, bm2, wc1, bc1, wc2, bc2, om, oc):
    pre = (a0[...] + a1[...] + hp[...]) * dinv[...] + b[...]
    h3 = jax.nn.relu(pre * (g[...] * BN_SCALE) + be[...])
    hm = jax.nn.relu(jnp.dot(h3, wm1[...],
                             preferred_element_type=jnp.float32) + bm1[...])
    m = jnp.dot(hm, wm2[...], preferred_element_type=jnp.float32) + bm2[...]
    om[...] = m[:, 0].reshape(1, 1, -1)
    hc = jax.nn.relu(jnp.dot(h3, wc1[...],
                             preferred_element_type=jnp.float32) + bc1[...])
    cg = jax.nn.sigmoid(
        jnp.dot(hc, wc2[...], preferred_element_type=jnp.float32) + bc2[...])
    oc[...] = cg[:, 0].reshape(1, 1, -1)


def _tc_ep3(a0, a1, hp, dinv, b3, g3, be3, Wm1, bm1, Wm2, bm2,
            Wc1, bc1, Wc2, bc2):
    def full(r, c):
        return pl.BlockSpec((r, c), lambda i: (0, 0))
    return pl.pallas_call(
        _ep3_body,
        grid=(NB,),
        in_specs=[
            pl.BlockSpec((BN, 64), lambda i: (i, 0)),
            pl.BlockSpec((BN, 64), lambda i: (i, 0)),
            pl.BlockSpec((BN, 64), lambda i: (i, 0)),
            pl.BlockSpec((BN, 1), lambda i: (i, 0)),
            full(1, 64), full(1, 64), full(1, 64),
            full(64, 32), full(1, 32), full(32, 1), full(1, 1),
            full(64, 32), full(1, 32), full(32, 1), full(1, 1),
        ],
        out_specs=[
            pl.BlockSpec((1, 1, BN), lambda i: (i, 0, 0)),
            pl.BlockSpec((1, 1, BN), lambda i: (i, 0, 0)),
        ],
        out_shape=[
            jax.ShapeDtypeStruct((NB, 1, BN), jnp.float32),
            jax.ShapeDtypeStruct((NB, 1, BN), jnp.float32),
        ],
    )(a0, a1, hp, dinv, b3.reshape(1, 64), g3.reshape(1, 64),
      be3.reshape(1, 64), Wm1, bm1.reshape(1, 32), Wm2, bm2.reshape(1, 1),
      Wc1, bc1.reshape(1, 32), Wc2, bc2.reshape(1, 1))


def kernel(x, edge_index, W1, b1, W2, b2, W3, b3, g1, be1, g2, be2, g3, be3,
           Wm1, bm1, Wm2, bm2, Wc1, bc1, Wc2, bc2):
    ei4 = edge_index.reshape(2, NW, NCH_A, CH)

    d0, d1 = _sc_deg(ei4)

    h1p, dinv = _tc_prep1(x, W1, d0, d1)

    a0, a1 = _mp96(h1p, ei4)
    q0, q1, q2, q3 = _tc_ep1(a0, a1, h1p, dinv, b1, g1, be1, W2)

    c0, c1, c2, c3 = _mp_feat64(q0, q1, q2, q3, ei4)
    h3p = _tc_ep2(c0, c1, c2, c3, q0, q1, q2, q3, dinv, b2, g2, be2, W3)

    e0, e1 = _mp64(h3p, ei4)
    motor, cog = _tc_ep3(e0, e1, h3p, dinv, b3, g3, be3,
                         Wm1, bm1, Wm2, bm2, Wc1, bc1, Wc2, bc2)
    return motor.reshape(N), cog.reshape(N)


# speedup vs baseline: 1.0885x; 1.0123x over previous
import functools

import jax
import jax.numpy as jnp
from jax import lax
from jax.experimental import pallas as pl
from jax.experimental.pallas import tpu as pltpu
from jax.experimental.pallas import tpu_sc as plsc

N = 10000
E = 320000
NC = 2
NS = 16
CH = 125
NW = NC * NS
EPW = E // NW
NCH_A = EPW // CH
NPA = 624
NREM = N - NS * NPA
DEGW = 16

BN = 2000
NB = N // BN

BN_SCALE = 1.0 / (1.0 + 1e-5) ** 0.5

_MESH = plsc.VectorSubcoreMesh(core_axis_name="c", subcore_axis_name="s")


def _per_tile_rows(s, do):
    do(pl.ds(s * NPA, NPA))

    @pl.when(s == NS - 1)
    def _():
        do(pl.ds(NS * NPA, NREM))


def _fill(ref, nrow, ncol, value):
    val = jnp.full((16,), value, jnp.float32)

    def row(i, carry):
        for k in range(ncol // 16):
            ref[i, pl.ds(k * 16, 16)] = val
        return carry

    lax.fori_loop(0, nrow, row, 0)


def _zero_shared(s, zbuf, accsh, d):
    _fill(zbuf, CH, d, 0.0)

    def piece(k, carry):
        pltpu.sync_copy(zbuf.at[pl.ds(0, 104)],
                        accsh.at[pl.ds(s * NPA + k * 104, 104)])
        return carry

    lax.fori_loop(0, NPA // 104, piece, 0)

    @pl.when(s == NS - 1)
    def _():
        pltpu.sync_copy(zbuf.at[pl.ds(0, NREM)],
                        accsh.at[pl.ds(NS * NPA, NREM)])


def _mp_pipelined(h_hbm, srcv, dstv, nch, rows, gsems, ssems, accsh):
    nbuf = len(rows)
    dummy = h_hbm.at[pl.ds(0, CH)]
    for b in range(nbuf):
        pltpu.async_copy(h_hbm.at[srcv.at[b]], rows[b], gsems[b])

    def round_(g, carry):
        for b in range(nbuf):
            j = nbuf * g + b
            bp = (b - 1) % nbuf

            def refill(j=j, bp=bp):
                pltpu.make_async_copy(dummy, rows[bp], ssems[bp]).wait()
                pltpu.async_copy(
                    h_hbm.at[srcv.at[jnp.minimum(j - 1 + nbuf, nch - 1)]],
                    rows[bp], gsems[bp])

            pltpu.make_async_copy(dummy, rows[b], gsems[b]).wait()
            pltpu.async_copy(rows[b], accsh.at[dstv.at[j]], ssems[b],
                             add=True)
            if b == 0:
                pl.when(g > 0)(refill)
            else:
                refill()
        return carry

    lax.fori_loop(0, nch // nbuf, round_, 0)
    for b in range(nbuf - 1):
        pltpu.make_async_copy(dummy, rows[b], gsems[b]).wait()
    pltpu.make_async_copy(dummy, rows[nbuf - 1], ssems[nbuf - 1]).wait()


def _deg_body(ei_hbm, deg0_hbm, deg1_hbm, dstv, onesv, degsh):
    c = lax.axis_index("c")
    s = lax.axis_index("s")
    wid = 2 * s + c
    pltpu.sync_copy(ei_hbm.at[1, wid], dstv)
    _zero_shared(s, onesv, degsh, DEGW)
    _fill(onesv, CH, DEGW, 1.0)
    plsc.subcore_barrier()

    def step(j, carry):
        pltpu.sync_copy(onesv, degsh.at[dstv.at[j]], add=True)
        return carry

    lax.fori_loop(0, NCH_A, step, 0)
    plsc.subcore_barrier()

    @pl.when(c == 0)
    def _():
        _per_tile_rows(s, lambda sl: pltpu.sync_copy(degsh.at[sl],
                                                     deg0_hbm.at[sl]))

    @pl.when(c == 1)
    def _():
        _per_tile_rows(s, lambda sl: pltpu.sync_copy(degsh.at[sl],
                                                     deg1_hbm.at[sl]))


_sc_deg = pl.kernel(
    _deg_body,
    out_type=(jax.ShapeDtypeStruct((N, DEGW), jnp.float32),
              jax.ShapeDtypeStruct((N, DEGW), jnp.float32)),
    mesh=_MESH,
    compiler_params=pltpu.CompilerParams(use_tc_tiling_on_sc=False),
    scratch_types=[
        pltpu.VMEM((NCH_A, CH), jnp.int32),
        pltpu.VMEM((CH, DEGW), jnp.float32),
        pltpu.VMEM_SHARED((N, DEGW), jnp.float32),
    ],
)


def _make_mp_edge(D):
    def body(h_hbm, ei_hbm, out0, out1,
             srcv, dstv, r0, r1, r2, r3, accsh,
             g0, g1, g2, g3, s0, s1, s2, s3):
        c = lax.axis_index("c")
        s = lax.axis_index("s")
        wid = 2 * s + c
        pltpu.sync_copy(ei_hbm.at[0, wid], srcv)
        pltpu.sync_copy(ei_hbm.at[1, wid], dstv)
        _zero_shared(s, r0, accsh, D)
        plsc.subcore_barrier()
        _mp_pipelined(h_hbm, srcv, dstv, NCH_A, [r0, r1, r2, r3],
                      [g0, g1, g2, g3], [s0, s1, s2, s3], accsh)
        plsc.subcore_barrier()

        @pl.when(c == 0)
        def _():
            _per_tile_rows(s, lambda sl: pltpu.sync_copy(accsh.at[sl],
                                                         out0.at[sl]))

        @pl.when(c == 1)
        def _():
            _per_tile_rows(s, lambda sl: pltpu.sync_copy(accsh.at[sl],
                                                         out1.at[sl]))

    return pl.kernel(
        body,
        out_type=(jax.ShapeDtypeStruct((N, D), jnp.float32),
                  jax.ShapeDtypeStruct((N, D), jnp.float32)),
        mesh=_MESH,
        compiler_params=pltpu.CompilerParams(use_tc_tiling_on_sc=False),
        scratch_types=(
            [pltpu.VMEM((NCH_A, CH), jnp.int32),
             pltpu.VMEM((NCH_A, CH), jnp.int32)]
            + [pltpu.VMEM((CH, D), jnp.float32)] * 4
            + [pltpu.VMEM_SHARED((N, D), jnp.float32)]
            + [pltpu.SemaphoreType.DMA] * 8
        ),
    )


def _make_mp_feat(Dq):
    def body(hq0, hq1, hq2, hq3, ei_hbm,
             o0, o1, o2, o3,
             srcv, dstv, r0, r1, r2, r3, accsh,
             g0, g1, g2, g3, s0, s1, s2, s3):
        c = lax.axis_index("c")
        s = lax.axis_index("s")

        def phase(h_hbm, out_hbm):
            _zero_shared(s, r0, accsh, Dq)
            plsc.subcore_barrier()

            for q in range(2):
                pltpu.sync_copy(ei_hbm.at[0, 2 * s + q], srcv)
                pltpu.sync_copy(ei_hbm.at[1, 2 * s + q], dstv)
                _mp_pipelined(h_hbm, srcv, dstv, NCH_A, [r0, r1, r2, r3],
                              [g0, g1, g2, g3], [s0, s1, s2, s3], accsh)
            plsc.subcore_barrier()
            _per_tile_rows(s, lambda sl: pltpu.sync_copy(accsh.at[sl],
                                                         out_hbm.at[sl]))
            plsc.subcore_barrier()

        @pl.when(c == 0)
        def _():
            phase(hq0, o0)
            phase(hq1, o1)

        @pl.when(c == 1)
        def _():
            phase(hq2, o2)
            phase(hq3, o3)

    return pl.kernel(
        body,
        out_type=tuple(jax.ShapeDtypeStruct((N, Dq), jnp.float32)
                       for _ in range(4)),
        mesh=_MESH,
        compiler_params=pltpu.CompilerParams(use_tc_tiling_on_sc=False),
        scratch_types=(
            [pltpu.VMEM((NCH_A, CH), jnp.int32),
             pltpu.VMEM((NCH_A, CH), jnp.int32)]
            + [pltpu.VMEM((CH, Dq), jnp.float32)] * 4
            + [pltpu.VMEM_SHARED((N, Dq), jnp.float32)]
            + [pltpu.SemaphoreType.DMA] * 8
        ),
    )


_mp96 = _make_mp_edge(96)
_mp64 = _make_mp_edge(64)
_mp_feat64 = _make_mp_feat(64)


def _prep1_body(x_ref, w_ref, d0_ref, d1_ref, o_ref, odinv_ref):
    dinv = lax.rsqrt(d0_ref[...][:, :1] + d1_ref[...][:, :1] + 1.0)
    h = jnp.dot(x_ref[...], w_ref[...], preferred_element_type=jnp.float32)
    o_ref[...] = h * dinv
    odinv_ref[...] = dinv


def _tc_prep1(x, W1, d0, d1):
    return pl.pallas_call(
        _prep1_body,
        grid=(NB,),
        in_specs=[
            pl.BlockSpec((BN, 128), lambda i: (i, 0)),
            pl.BlockSpec((128, 96), lambda i: (0, 0)),
            pl.BlockSpec((BN, DEGW), lambda i: (i, 0)),
            pl.BlockSpec((BN, DEGW), lambda i: (i, 0)),
        ],
        out_specs=[
            pl.BlockSpec((BN, 96), lambda i: (i, 0)),
            pl.BlockSpec((BN, 1), lambda i: (i, 0)),
        ],
        out_shape=[
            jax.ShapeDtypeStruct((N, 96), jnp.float32),
            jax.ShapeDtypeStruct((N, 1), jnp.float32),
        ],
    )(x, W1, d0, d1)


def _ep1_body(a0, a1, hp, dinv, b, g, be, w, o0, o1, o2, o3):
    pre = (a0[...] + a1[...] + hp[...]) * dinv[...] + b[...]
    h1 = jax.nn.relu(pre * (g[...] * BN_SCALE) + be[...])
    h2 = jnp.dot(h1, w[...], preferred_element_type=jnp.float32) * dinv[...]
    o0[...] = h2[:, 0:64]
    o1[...] = h2[:, 64:128]
    o2[...] = h2[:, 128:192]
    o3[...] = h2[:, 192:256]


def _tc_ep1(a0, a1, hp, dinv, b1, g1, be1, W2):
    return pl.pallas_call(
        _ep1_body,
        grid=(NB,),
        in_specs=[
            pl.BlockSpec((BN, 96), lambda i: (i, 0)),
            pl.BlockSpec((BN, 96), lambda i: (i, 0)),
            pl.BlockSpec((BN, 96), lambda i: (i, 0)),
            pl.BlockSpec((BN, 1), lambda i: (i, 0)),
            pl.BlockSpec((1, 96), lambda i: (0, 0)),
            pl.BlockSpec((1, 96), lambda i: (0, 0)),
            pl.BlockSpec((1, 96), lambda i: (0, 0)),
            pl.BlockSpec((96, 256), lambda i: (0, 0)),
        ],
        out_specs=[pl.BlockSpec((BN, 64), lambda i: (i, 0))] * 4,
        out_shape=[jax.ShapeDtypeStruct((N, 64), jnp.float32)] * 4,
    )(a0, a1, hp, dinv, b1.reshape(1, 96), g1.reshape(1, 96),
      be1.reshape(1, 96), W2)


def _ep2_body(a0, a1, a2, a3, hp0, hp1, hp2, hp3, dinv, b, g, be, w, o_ref):
    accf = jnp.concatenate([a0[...] + hp0[...], a1[...] + hp1[...],
                            a2[...] + hp2[...], a3[...] + hp3[...]], axis=1)
    pre = accf * dinv[...] + b[...]
    h2 = jax.nn.relu(pre * (g[...] * BN_SCALE) + be[...])
    o_ref[...] = jnp.dot(h2, w[...],
                         preferred_element_type=jnp.float32) * dinv[...]


def _tc_ep2(a0, a1, a2, a3, hp0, hp1, hp2, hp3, dinv, b2, g2, be2, W3):
    return pl.pallas_call(
        _ep2_body,
        grid=(NB,),
        in_specs=[
            pl.BlockSpec((BN, 64), lambda i: (i, 0)),
            pl.BlockSpec((BN, 64), lambda i: (i, 0)),
            pl.BlockSpec((BN, 64), lambda i: (i, 0)),
            pl.BlockSpec((BN, 64), lambda i: (i, 0)),
            pl.BlockSpec((BN, 64), lambda i: (i, 0)),
            pl.BlockSpec((BN, 64), lambda i: (i, 0)),
            pl.BlockSpec((BN, 64), lambda i: (i, 0)),
            pl.BlockSpec((BN, 64), lambda i: (i, 0)),
            pl.BlockSpec((BN, 1), lambda i: (i, 0)),
            pl.BlockSpec((1, 256), lambda i: (0, 0)),
            pl.BlockSpec((1, 256), lambda i: (0, 0)),
            pl.BlockSpec((1, 256), lambda i: (0, 0)),
            pl.BlockSpec((256, 64), lambda i: (0, 0)),
        ],
        out_specs=pl.BlockSpec((BN, 64), lambda i: (i, 0)),
        out_shape=jax.ShapeDtypeStruct((N, 64), jnp.float32),
    )(a0, a1, a2, a3, hp0, hp1, hp2, hp3, dinv, b2.reshape(1, 256),
      g2.reshape(1, 256), be2.reshape(1, 256), W3)


def _ep3_body(a0, a1, hp, dinv, b, g, be,
              wm1, bm1, wm2, bm2, wc1, bc1, wc2, bc2, om, oc):
    pre = (a0[...] + a1[...] + hp[...]) * dinv[...] + b[...]
    h3 = jax.nn.relu(pre * (g[...] * BN_SCALE) + be[...])
    hm = jax.nn.relu(jnp.dot(h3, wm1[...],
                             preferred_element_type=jnp.float32) + bm1[...])
    m = jnp.dot(hm, wm2[...], preferred_element_type=jnp.float32) + bm2[...]
    om[...] = m[:, 0].reshape(1, 1, -1)
    hc = jax.nn.relu(jnp.dot(h3, wc1[...],
                             preferred_element_type=jnp.float32) + bc1[...])
    cg = jax.nn.sigmoid(
        jnp.dot(hc, wc2[...], preferred_element_type=jnp.float32) + bc2[...])
    oc[...] = cg[:, 0].reshape(1, 1, -1)


def _tc_ep3(a0, a1, hp, dinv, b3, g3, be3, Wm1, bm1, Wm2, bm2,
            Wc1, bc1, Wc2, bc2):
    def full(r, c):
        return pl.BlockSpec((r, c), lambda i: (0, 0))
    return pl.pallas_call(
        _ep3_body,
        grid=(NB,),
        in_specs=[
            pl.BlockSpec((BN, 64), lambda i: (i, 0)),
            pl.BlockSpec((BN, 64), lambda i: (i, 0)),
            pl.BlockSpec((BN, 64), lambda i: (i, 0)),
            pl.BlockSpec((BN, 1), lambda i: (i, 0)),
            full(1, 64), full(1, 64), full(1, 64),
            full(64, 32), full(1, 32), full(32, 1), full(1, 1),
            full(64, 32), full(1, 32), full(32, 1), full(1, 1),
        ],
        out_specs=[
            pl.BlockSpec((1, 1, BN), lambda i: (i, 0, 0)),
            pl.BlockSpec((1, 1, BN), lambda i: (i, 0, 0)),
        ],
        out_shape=[
            jax.ShapeDtypeStruct((NB, 1, BN), jnp.float32),
            jax.ShapeDtypeStruct((NB, 1, BN), jnp.float32),
        ],
    )(a0, a1, hp, dinv, b3.reshape(1, 64), g3.reshape(1, 64),
      be3.reshape(1, 64), Wm1, bm1.reshape(1, 32), Wm2, bm2.reshape(1, 1),
      Wc1, bc1.reshape(1, 32), Wc2, bc2.reshape(1, 1))


def kernel(x, edge_index, W1, b1, W2, b2, W3, b3, g1, be1, g2, be2, g3, be3,
           Wm1, bm1, Wm2, bm2, Wc1, bc1, Wc2, bc2):
    ei4 = edge_index.reshape(2, NW, NCH_A, CH)

    d0, d1 = _sc_deg(ei4)

    h1p, dinv = _tc_prep1(x, W1, d0, d1)

    a0, a1 = _mp96(h1p, ei4)
    q0, q1, q2, q3 = _tc_ep1(a0, a1, h1p, dinv, b1, g1, be1, W2)

    c0, c1, c2, c3 = _mp_feat64(q0, q1, q2, q3, ei4)
    h3p = _tc_ep2(c0, c1, c2, c3, q0, q1, q2, q3, dinv, b2, g2, be2, W3)

    e0, e1 = _mp64(h3p, ei4)
    motor, cog = _tc_ep3(e0, e1, h3p, dinv, b3, g3, be3,
                         Wm1, bm1, Wm2, bm2, Wc1, bc1, Wc2, bc2)
    return motor.reshape(N), cog.reshape(N)
